# Initial kernel scaffold; baseline (speedup 1.0000x reference)
import jax
import jax.numpy as jnp
from jax.experimental import pallas as pl


def kernel(inputs_0, inputs_1, inputs_2, inputs_3, inputs_4, inputs_5, inputs_6, inputs_7, inputs_8, inputs_9, inputs_10, inputs_11, inputs_12, inputs_13, inputs_14, inputs_15, inputs_16, inputs_17, inputs_18, inputs_19, inputs_20, inputs_21, inputs_22, inputs_23, inputs_24, inputs_25, table_0, table_1, table_2, table_3, table_4, table_5, table_6, table_7, table_8, table_9, table_10, table_11, table_12, table_13, table_14, table_15, table_16, table_17, table_18, table_19, table_20, table_21, table_22, table_23, table_24, table_25):
    raise NotImplementedError("write your pallas kernel here")


# speedup vs baseline: 9.8938x; 9.8938x over previous
import functools

import jax
import jax.numpy as jnp
from jax import lax
from jax.experimental import pallas as pl
from jax.experimental.pallas import tpu as pltpu
from jax.experimental.pallas import tpu_sc as plsc

NUM_VARS = 26
VOCAB = 100002
DIM = 32
B = 16384
L = 20
N = B * L
NW = 32
ROWS_W = N // NW
C = 1024
NCHUNK = ROWS_W // C
LANES = 16

_mesh = plsc.VectorSubcoreMesh(core_axis_name="c", subcore_axis_name="s")


@functools.partial(
    pl.kernel,
    mesh=_mesh,
    out_type=jax.ShapeDtypeStruct((N, NUM_VARS * DIM), jnp.float32),
    scratch_types=[
        pltpu.VMEM((C,), jnp.int32),
        pltpu.VMEM((C, DIM), jnp.float32),
        pltpu.SemaphoreType.DMA,
    ],
    compiler_params=pltpu.CompilerParams(use_tc_tiling_on_sc=False),
)
def _bank(*refs):
    inputs = refs[:NUM_VARS]
    tables = refs[NUM_VARS:2 * NUM_VARS]
    out = refs[2 * NUM_VARS]
    idx_v, rows_v, sem = refs[2 * NUM_VARS + 1:]

    wid = lax.axis_index("s") * 2 + lax.axis_index("c")
    wbase = wid * ROWS_W

    for i in range(NUM_VARS):
        inp = inputs[i]
        tbl = tables[i]

        def chunk_body(ci, _, inp=inp, tbl=tbl, i=i):
            base = wbase + ci * C
            pltpu.sync_copy(inp.at[pl.ds(base, C)], idx_v)

            def clamp_body(j, _):
                v = idx_v[pl.ds(j * LANES, LANES)]
                v = jnp.where(v == -1, VOCAB - 1, v)
                v = jnp.minimum(jnp.maximum(v, 0), VOCAB - 1)
                idx_v[pl.ds(j * LANES, LANES)] = v
                return _

            lax.fori_loop(0, C // LANES, clamp_body, None)

            pltpu.async_copy(tbl.at[idx_v], rows_v, sem).wait()
            pltpu.sync_copy(rows_v, out.at[pl.ds(base, C), pl.ds(i * DIM, DIM)])
            return _

        lax.fori_loop(0, NCHUNK, chunk_body, None)


def kernel(inputs_0, inputs_1, inputs_2, inputs_3, inputs_4, inputs_5, inputs_6, inputs_7, inputs_8, inputs_9, inputs_10, inputs_11, inputs_12, inputs_13, inputs_14, inputs_15, inputs_16, inputs_17, inputs_18, inputs_19, inputs_20, inputs_21, inputs_22, inputs_23, inputs_24, inputs_25, table_0, table_1, table_2, table_3, table_4, table_5, table_6, table_7, table_8, table_9, table_10, table_11, table_12, table_13, table_14, table_15, table_16, table_17, table_18, table_19, table_20, table_21, table_22, table_23, table_24, table_25):
    args = locals()
    flats = [args[f"inputs_{i}"].reshape(N) for i in range(NUM_VARS)]
    tabs = [args[f"table_{i}"] for i in range(NUM_VARS)]
    out = _bank(*flats, *tabs)
    return out.reshape(B, L, NUM_VARS * DIM)


# speedup vs baseline: 10.4162x; 1.0528x over previous
import functools

import jax
import jax.numpy as jnp
from jax import lax
from jax.experimental import pallas as pl
from jax.experimental.pallas import tpu as pltpu
from jax.experimental.pallas import tpu_sc as plsc

NUM_VARS = 26
VOCAB = 100002
DIM = 32
B = 16384
L = 20
N = B * L
NW = 32
ROWS_W = N // NW
C = 1024
NCHUNK = ROWS_W // C
LANES = 16
R = 2

_mesh = plsc.VectorSubcoreMesh(core_axis_name="c", subcore_axis_name="s")


@functools.partial(
    pl.kernel,
    mesh=_mesh,
    out_type=jax.ShapeDtypeStruct((N, NUM_VARS * DIM), jnp.float32),
    scratch_types=[
        pltpu.VMEM((R, C), jnp.int32),
        pltpu.VMEM((R, C, DIM), jnp.float32),
        pltpu.SemaphoreType.DMA((R,)),
        pltpu.SemaphoreType.DMA((R,)),
    ],
    compiler_params=pltpu.CompilerParams(use_tc_tiling_on_sc=False),
)
def _bank(*refs):
    inputs = refs[:NUM_VARS]
    tables = refs[NUM_VARS:2 * NUM_VARS]
    out = refs[2 * NUM_VARS]
    idx_v, rows_v, gsem, ssem = refs[2 * NUM_VARS + 1:]

    wid = lax.axis_index("s") * 2 + lax.axis_index("c")
    wbase = wid * ROWS_W

    def chunk_body(ci, _):
        base = wbase + ci * C

        def load_clamp_gather(s):
            r = s % R
            pltpu.sync_copy(inputs[s].at[pl.ds(base, C)], idx_v.at[r])

            def clamp_body(j, _):
                v = idx_v[r, pl.ds(j * LANES, LANES)]
                v = jnp.where(v == -1, VOCAB - 1, v)
                v = jnp.minimum(jnp.maximum(v, 0), VOCAB - 1)
                idx_v[r, pl.ds(j * LANES, LANES)] = v
                return _

            lax.fori_loop(0, C // LANES, clamp_body, None)
            pltpu.async_copy(tables[s].at[idx_v.at[r]], rows_v.at[r], gsem.at[r])

        def scatter(s):
            r = s % R
            pltpu.make_async_copy(tables[s].at[idx_v.at[r]], rows_v.at[r],
                                  gsem.at[r]).wait()
            pltpu.async_copy(rows_v.at[r],
                             out.at[pl.ds(base, C), pl.ds(s * DIM, DIM)],
                             ssem.at[r])

        def drain_scatter(s):
            r = s % R
            pltpu.make_async_copy(rows_v.at[r],
                                  out.at[pl.ds(base, C), pl.ds(s * DIM, DIM)],
                                  ssem.at[r]).wait()

        for s in range(NUM_VARS):
            if s >= R:
                drain_scatter(s - R)
            load_clamp_gather(s)
            if s >= 1:
                scatter(s - 1)
        scatter(NUM_VARS - 1)
        drain_scatter(NUM_VARS - 2)
        drain_scatter(NUM_VARS - 1)
        return _

    lax.fori_loop(0, NCHUNK, chunk_body, None)


def kernel(inputs_0, inputs_1, inputs_2, inputs_3, inputs_4, inputs_5, inputs_6, inputs_7, inputs_8, inputs_9, inputs_10, inputs_11, inputs_12, inputs_13, inputs_14, inputs_15, inputs_16, inputs_17, inputs_18, inputs_19, inputs_20, inputs_21, inputs_22, inputs_23, inputs_24, inputs_25, table_0, table_1, table_2, table_3, table_4, table_5, table_6, table_7, table_8, table_9, table_10, table_11, table_12, table_13, table_14, table_15, table_16, table_17, table_18, table_19, table_20, table_21, table_22, table_23, table_24, table_25):
    args = locals()
    flats = [args[f"inputs_{i}"].reshape(N) for i in range(NUM_VARS)]
    tabs = [args[f"table_{i}"] for i in range(NUM_VARS)]
    out = _bank(*flats, *tabs)
    return out.reshape(B, L, NUM_VARS * DIM)
